# Initial kernel scaffold; baseline (speedup 1.0000x reference)
import jax
import jax.numpy as jnp
from jax.experimental import pallas as pl


def kernel(proposals, cls_preds, offset_preds):
    raise NotImplementedError("write your pallas kernel here")


# speedup vs baseline: 2.5919x; 2.5919x over previous
import functools

import jax
import jax.numpy as jnp
from jax import lax
from jax.experimental import pallas as pl
from jax.experimental.pallas import tpu as pltpu
from jax.experimental.pallas import tpu_sc as plsc

_BUCKETS = 14
_SIDE = 7
_SCALE = 1.7
_BP = 800
_GP = _BP // 16
_NC = 2
_NS = 16
_NW = _NC * _NS


@functools.lru_cache(maxsize=None)
def _build(P):
    assert P % _BP == 0
    NB = P // _BP
    mesh = plsc.VectorSubcoreMesh(core_axis_name="c", subcore_axis_name="s")

    @functools.partial(
        pl.kernel,
        mesh=mesh,
        compiler_params=pltpu.CompilerParams(needs_layout_passes=False),
        out_type=[
            jax.ShapeDtypeStruct((P * 4,), jnp.float32),
            jax.ShapeDtypeStruct((P,), jnp.float32),
        ],
        scratch_types=[
            pltpu.VMEM((_BP * 28,), jnp.float32),
            pltpu.VMEM((_BP * 28,), jnp.float32),
            pltpu.VMEM((_BP * 4,), jnp.float32),
            pltpu.VMEM((_BP * 4,), jnp.float32),
            pltpu.VMEM((_BP,), jnp.float32),
        ],
    )
    def run(cls_hbm, off_hbm, prop_hbm, bbox_hbm, conf_hbm,
            cls_vm, off_vm, prop_vm, bbox_vm, conf_vm):
        wid = lax.axis_index("s") * _NC + lax.axis_index("c")
        nb = NB // _NW + jnp.where(wid < NB % _NW, 1, 0)
        iota = lax.iota(jnp.int32, 16)
        i28 = iota * 28
        i4 = iota * 4

        def group_body(g, carry):
            base28 = i28 + g * (16 * 28)
            base4 = i4 + g * (16 * 4)
            base16 = iota + g * 16

            def side(s):
                c = [plsc.load_gather(cls_vm, [base28 + (s * 7 + k)])
                     for k in range(7)]
                v1 = c[0]
                i1 = jnp.zeros((16,), jnp.float32)
                v2 = jnp.full((16,), -jnp.inf, jnp.float32)
                i2 = jnp.zeros((16,), jnp.float32)
                for k in range(1, 7):
                    kf = jnp.float32(k)
                    gt1 = c[k] > v1
                    gt2 = c[k] > v2
                    nv2 = jnp.where(gt2, c[k], v2)
                    v2 = jnp.where(gt1, v1, nv2)
                    ni2 = jnp.where(gt2, kf, i2)
                    i2 = jnp.where(gt1, i1, ni2)
                    v1 = jnp.where(gt1, c[k], v1)
                    i1 = jnp.where(gt1, kf, i1)
                z = jnp.exp(c[0] - v1)
                for k in range(1, 7):
                    z = z + jnp.exp(c[k] - v1)
                p1 = 1.0 / z
                p2 = jnp.exp(v2 - v1) * p1
                conf_s = p1 + p2 * (jnp.abs(i1 - i2) - 1.0)
                o = plsc.load_gather(
                    off_vm, [base28 + (s * 7) + i1.astype(jnp.int32)])
                return i1, o, conf_s

            il, ol, cl = side(0)
            ir, orr, cr = side(1)
            it, ot, ct = side(2)
            idd, od, cd = side(3)

            x1 = plsc.load_gather(prop_vm, [base4 + 0])
            y1 = plsc.load_gather(prop_vm, [base4 + 1])
            x2 = plsc.load_gather(prop_vm, [base4 + 2])
            y2 = plsc.load_gather(prop_vm, [base4 + 3])
            cx = (x1 + x2) * 0.5
            cy = (y1 + y2) * 0.5
            w = (x2 - x1) * _SCALE
            h = (y2 - y1) * _SCALE
            px1 = cx - 0.5 * w
            px2 = cx + 0.5 * w
            py1 = cy - 0.5 * h
            py2 = cy + 0.5 * h
            bw = (px2 - px1) * (1.0 / _BUCKETS)
            bh = (py2 - py1) * (1.0 / _BUCKETS)
            x1o = px1 + (0.5 + il) * bw - ol * bw
            x2o = px2 - (0.5 + ir) * bw - orr * bw
            y1o = py1 + (0.5 + it) * bh - ot * bh
            y2o = py2 - (0.5 + idd) * bh - od * bh
            conf = (cl + cr + ct + cd) * 0.25
            plsc.store_scatter(bbox_vm, [base4 + 0], x1o)
            plsc.store_scatter(bbox_vm, [base4 + 1], y1o)
            plsc.store_scatter(bbox_vm, [base4 + 2], x2o)
            plsc.store_scatter(bbox_vm, [base4 + 3], y2o)
            plsc.store_scatter(conf_vm, [base16], conf)
            return carry

        def block_body(j, carry):
            k = wid + j * _NW
            pltpu.sync_copy(cls_hbm.at[pl.ds(k * (_BP * 28), _BP * 28)], cls_vm)
            pltpu.sync_copy(off_hbm.at[pl.ds(k * (_BP * 28), _BP * 28)], off_vm)
            pltpu.sync_copy(prop_hbm.at[pl.ds(k * (_BP * 4), _BP * 4)], prop_vm)
            lax.fori_loop(0, _GP, group_body, 0)
            pltpu.sync_copy(bbox_vm, bbox_hbm.at[pl.ds(k * (_BP * 4), _BP * 4)])
            pltpu.sync_copy(conf_vm, conf_hbm.at[pl.ds(k * _BP, _BP)])
            return carry

        lax.fori_loop(0, nb, block_body, 0)

    return run


@jax.jit
def kernel(proposals, cls_preds, offset_preds):
    B, N, _ = proposals.shape
    P = B * N
    run = _build(P)
    bbox_flat, conf_flat = run(
        cls_preds.reshape(P * 4 * _SIDE),
        offset_preds.reshape(P * 4 * _SIDE),
        proposals.reshape(P * 4),
    )
    return bbox_flat.reshape(B, N, 4), conf_flat.reshape(B, N)
